# Initial kernel scaffold; baseline (speedup 1.0000x reference)
import functools

import jax
import jax.numpy as jnp
from jax import lax
from jax.experimental import pallas as pl
from jax.experimental.pallas import tpu as pltpu
from jax.experimental.pallas import tpu_sc as plsc

BATCH = 1024
TIME = 50
D = 1000
N = BATCH * TIME
NC, NS = 2, 16
NW = NC * NS
PER_W = N // NW
CHUNK = 40
NCHUNK = PER_W // CHUNK

_mesh = plsc.VectorSubcoreMesh(core_axis_name="c", subcore_axis_name="s")


@functools.partial(
    pl.kernel,
    mesh=_mesh,
    out_type=jax.ShapeDtypeStruct((N, D), jnp.float32),
    scratch_types=[
        pltpu.VMEM((NCHUNK, CHUNK), jnp.int32),
        pltpu.VMEM((CHUNK, D), jnp.float32),
        pltpu.VMEM((CHUNK, D), jnp.float32),
        pltpu.SemaphoreType.DMA,
        pltpu.SemaphoreType.DMA,
        pltpu.SemaphoreType.DMA,
        pltpu.SemaphoreType.DMA,
    ],
)
def _embed(idx_hbm, table_hbm, out_hbm, idx_v, buf0, buf1, g0, g1, w0, w1):
    wid = lax.axis_index("s") * NC + lax.axis_index("c")
    base = wid * PER_W

    pltpu.sync_copy(idx_hbm.at[wid], idx_v)

    def gather_start(c, buf, sem):
        return pltpu.async_copy(table_hbm.at[idx_v.at[c]], buf, sem)

    def gather_wait(c, buf, sem):
        pltpu.make_async_copy(table_hbm.at[idx_v.at[c]], buf, sem).wait()

    def write_start(c, buf, sem):
        dst = out_hbm.at[pl.ds(base + c * CHUNK, CHUNK)]
        return pltpu.async_copy(buf, dst, sem)

    def write_wait(c, buf, sem):
        dst = out_hbm.at[pl.ds(base + c * CHUNK, CHUNK)]
        pltpu.make_async_copy(buf, dst, sem).wait()

    gather_start(0, buf0, g0)
    gather_start(1, buf1, g1)

    def body(j, carry):
        c0 = 2 * j
        gather_wait(c0, buf0, g0)
        write_start(c0, buf0, w0)
        gather_wait(c0 + 1, buf1, g1)
        write_start(c0 + 1, buf1, w1)
        write_wait(c0, buf0, w0)
        gather_start(c0 + 2, buf0, g0)
        write_wait(c0 + 1, buf1, w1)
        gather_start(c0 + 3, buf1, g1)
        return carry

    lax.fori_loop(0, NCHUNK // 2 - 1, body, 0)

    cL = NCHUNK - 2
    gather_wait(cL, buf0, g0)
    hw0 = write_start(cL, buf0, w0)
    gather_wait(cL + 1, buf1, g1)
    hw1 = write_start(cL + 1, buf1, w1)
    hw0.wait()
    hw1.wait()


def kernel(idx, table):
    idx_r = idx.reshape(NW, NCHUNK, CHUNK).astype(jnp.int32)
    out = _embed(idx_r, table)
    return out.reshape(BATCH, TIME, D)


# speedup vs baseline: 1.0198x; 1.0198x over previous
import functools

import jax
import jax.numpy as jnp
from jax import lax
from jax.experimental import pallas as pl
from jax.experimental.pallas import tpu as pltpu
from jax.experimental.pallas import tpu_sc as plsc

BATCH = 1024
TIME = 50
D = 1000
N = BATCH * TIME
NC, NS = 2, 16
NW = NC * NS
PER_W = N // NW
CHUNK = 40
NCHUNK = PER_W // CHUNK

_mesh = plsc.VectorSubcoreMesh(core_axis_name="c", subcore_axis_name="s")


@functools.partial(
    pl.kernel,
    mesh=_mesh,
    out_type=jax.ShapeDtypeStruct((N, D), jnp.float32),
    scratch_types=[
        pltpu.VMEM((NCHUNK, CHUNK), jnp.int32),
        pltpu.VMEM((CHUNK, D), jnp.float32),
        pltpu.VMEM((CHUNK, D), jnp.float32),
        pltpu.SemaphoreType.DMA,
        pltpu.SemaphoreType.DMA,
        pltpu.SemaphoreType.DMA,
        pltpu.SemaphoreType.DMA,
    ],
    compiler_params=pltpu.CompilerParams(use_tc_tiling_on_sc=False),
)
def _embed(idx_hbm, table_hbm, out_hbm, idx_v, buf0, buf1, g0, g1, w0, w1):
    wid = lax.axis_index("s") * NC + lax.axis_index("c")
    base = wid * PER_W

    pltpu.sync_copy(idx_hbm.at[wid], idx_v)

    def gather_start(c, buf, sem):
        return pltpu.async_copy(table_hbm.at[idx_v.at[c]], buf, sem)

    def gather_wait(c, buf, sem):
        pltpu.make_async_copy(table_hbm.at[idx_v.at[c]], buf, sem).wait()

    def write_start(c, buf, sem):
        dst = out_hbm.at[pl.ds(base + c * CHUNK, CHUNK)]
        return pltpu.async_copy(buf, dst, sem)

    def write_wait(c, buf, sem):
        dst = out_hbm.at[pl.ds(base + c * CHUNK, CHUNK)]
        pltpu.make_async_copy(buf, dst, sem).wait()

    gather_start(0, buf0, g0)
    gather_start(1, buf1, g1)

    def body(j, carry):
        c0 = 2 * j
        gather_wait(c0, buf0, g0)
        write_start(c0, buf0, w0)
        gather_wait(c0 + 1, buf1, g1)
        write_start(c0 + 1, buf1, w1)
        write_wait(c0, buf0, w0)
        gather_start(c0 + 2, buf0, g0)
        write_wait(c0 + 1, buf1, w1)
        gather_start(c0 + 3, buf1, g1)
        return carry

    lax.fori_loop(0, NCHUNK // 2 - 1, body, 0)

    cL = NCHUNK - 2
    gather_wait(cL, buf0, g0)
    hw0 = write_start(cL, buf0, w0)
    gather_wait(cL + 1, buf1, g1)
    hw1 = write_start(cL + 1, buf1, w1)
    hw0.wait()
    hw1.wait()


def kernel(idx, table):
    idx_r = idx.reshape(NW, NCHUNK, CHUNK).astype(jnp.int32)
    out = _embed(idx_r, table)
    return out.reshape(BATCH, TIME, D)
